# Initial kernel scaffold; baseline (speedup 1.0000x reference)
import jax
import jax.numpy as jnp
from jax.experimental import pallas as pl


def kernel(x, idx0, idx1, idx2, idx3, idx4, w0, w1, w2, w3, w4):
    raise NotImplementedError("write your pallas kernel here")


# speedup vs baseline: 23.7672x; 23.7672x over previous
import functools

import jax
import jax.numpy as jnp
from jax.experimental import pallas as pl

B = 16384
N_IN = 256
SIZES = (128, 128, 128, 128, 64)
TOTALS = (256, 384, 512, 640, 768)
K = 16
BLK = 1024


def _densify_body(idx0, idx1, idx2, idx3, idx4, w0, w1, w2, w3, w4,
                  m0, m1, m2, m3, m4):
    idx_refs = (idx0, idx1, idx2, idx3, idx4)
    w_refs = (w0, w1, w2, w3, w4)
    m_refs = (m0, m1, m2, m3, m4)
    for li in range(5):
        sz = SIZES[li]
        rows = TOTALS[li]
        idx = idx_refs[li][...]
        idx = jnp.where(idx < N_IN, N_IN - 1 - idx, idx)
        w = w_refs[li][...]
        row_id = jax.lax.broadcasted_iota(jnp.int32, (rows, sz), 0)
        m = jnp.zeros((rows, sz), dtype=jnp.float32)
        for k in range(K):
            m = m + jnp.where(row_id == idx[k][None, :],
                              w[k][None, :], 0.0)
        m_refs[li][...] = m


def _forward_body(x_ref, m0, m1, m2, m3, m4, out_ref):
    dot = functools.partial(jnp.dot, preferred_element_type=jnp.float32)
    x = x_ref[...]
    h0 = jnp.tanh(dot(x, m0[...]))
    h1 = jax.nn.relu(dot(x, m1[:256]) + dot(h0, m1[256:]))
    h2 = jax.nn.sigmoid(dot(x, m2[:256]) + dot(h0, m2[256:384])
                        + dot(h1, m2[384:]))
    h3 = jnp.tanh(dot(x, m3[:256]) + dot(h0, m3[256:384])
                  + dot(h1, m3[384:512]) + dot(h2, m3[512:]))
    out_ref[...] = (dot(x, m4[:256]) + dot(h0, m4[256:384])
                    + dot(h1, m4[384:512]) + dot(h2, m4[512:640])
                    + dot(h3, m4[640:]))


def kernel(x, idx0, idx1, idx2, idx3, idx4, w0, w1, w2, w3, w4):
    idxs = [a.T for a in (idx0, idx1, idx2, idx3, idx4)]
    ws = [a.T for a in (w0, w1, w2, w3, w4)]

    ms = pl.pallas_call(
        _densify_body,
        out_shape=[jax.ShapeDtypeStruct((TOTALS[li], SIZES[li]), jnp.float32)
                   for li in range(5)],
    )(*idxs, *ws)

    grid = (B // BLK,)
    out = pl.pallas_call(
        _forward_body,
        grid=grid,
        in_specs=[pl.BlockSpec((BLK, N_IN), lambda i: (i, 0))]
        + [pl.BlockSpec((TOTALS[li], SIZES[li]), lambda i: (0, 0))
           for li in range(5)],
        out_specs=pl.BlockSpec((BLK, SIZES[-1]), lambda i: (i, 0)),
        out_shape=jax.ShapeDtypeStruct((B, SIZES[-1]), jnp.float32),
    )(x, *ms)
    return out
